# Initial kernel scaffold; baseline (speedup 1.0000x reference)
import jax
import jax.numpy as jnp
from jax.experimental import pallas as pl


def kernel(x, edge_index, batch, base, bn_gamma, bn_beta, W1, b1, W2, b2, W3, b3, gate_W, gate_b, fc1_W, fc1_b, fc2_W, fc2_b):
    raise NotImplementedError("write your pallas kernel here")


# speedup vs baseline: 14.4989x; 14.4989x over previous
import functools

import jax
import jax.numpy as jnp
from jax import lax
from jax.experimental import pallas as pl
from jax.experimental.pallas import tpu as pltpu
from jax.experimental.pallas import tpu_sc as plsc

N = 100000
G = 512
DIM = 128
F = 8
TN = 1024
T = 98
N_PAD = T * TN

EROW = 128
KB = 80
NW = 32
NSUB = 16
ROWS_PER_TILE = N_PAD // NSUB

_HIGHEST = jax.lax.Precision.HIGHEST


def _dot0(a, b):
    return lax.dot_general(a, b, (((0,), (0,)), ((), ())),
                           precision=_HIGHEST)


def _sc_agg(xp, xhalf, srcm, dstm):
    n_rows = srcm.shape[0]
    rows_per_w = n_rows // NW
    nblk = rows_per_w // KB

    mesh = plsc.VectorSubcoreMesh(core_axis_name="c", subcore_axis_name="s")

    @functools.partial(
        pl.kernel,
        out_type=jax.ShapeDtypeStruct((2 * N_PAD, F), jnp.float32),
        mesh=mesh,
        compiler_params=pltpu.CompilerParams(use_tc_tiling_on_sc=False),
        scratch_types=[
            pltpu.VMEM((KB, EROW), jnp.int32),
            pltpu.VMEM((KB, EROW), jnp.int32),
            pltpu.VMEM((EROW, F), jnp.float32),
            pltpu.VMEM((ROWS_PER_TILE, F), jnp.float32),
            pltpu.VMEM_SHARED((N_PAD, F), jnp.float32),
            pltpu.SemaphoreType.DMA,
        ],
    )
    def k(x_hbm, xh_hbm, src_hbm, dst_hbm, out_hbm,
          sbuf, dbuf, rows, bounce, acc_sh, sem):
        c = lax.axis_index("c")
        s = lax.axis_index("s")
        wid = c * NSUB + s

        r0 = s * ROWS_PER_TILE
        pltpu.sync_copy(xh_hbm.at[pl.ds(r0, ROWS_PER_TILE)], bounce)
        pltpu.sync_copy(bounce, acc_sh.at[pl.ds(r0, ROWS_PER_TILE)])
        plsc.subcore_barrier()

        wbase = wid * rows_per_w

        def blk_body(blk, carry):
            row0 = wbase + blk * KB
            pltpu.sync_copy(src_hbm.at[pl.ds(row0, KB)], sbuf)
            pltpu.sync_copy(dst_hbm.at[pl.ds(row0, KB)], dbuf)

            def edge_body(kk, carry2):
                pltpu.async_copy(x_hbm.at[sbuf.at[kk]], rows, sem).wait()
                pltpu.sync_copy(rows, acc_sh.at[dbuf.at[kk]], add=True)
                return carry2

            return lax.fori_loop(0, KB, edge_body, carry)

        lax.fori_loop(0, nblk, blk_body, 0)
        plsc.subcore_barrier()

        off = c * N_PAD + s * ROWS_PER_TILE
        pltpu.sync_copy(acc_sh.at[pl.ds(r0, ROWS_PER_TILE)], bounce)
        pltpu.sync_copy(bounce, out_hbm.at[pl.ds(off, ROWS_PER_TILE)])

    return k(xp, xhalf, srcm, dstm)


def _tc_body(p0_ref, p1_ref, bc_ref, W1_ref, b1_ref, W2_ref, b2_ref,
             W3_ref, b3_ref, gW_ref, gb_ref, f1W_ref, f1b_ref, f2W_ref,
             f2b_ref, gam_ref, bet_ref, base_ref,
             out_o_ref, out_a_ref, out_b_ref,
             s_ref, m_ref, d_ref, P_ref):
    p = pl.program_id(0)
    t = pl.program_id(1)

    @pl.when(p == 0)
    def _phase0():
        h3 = p0_ref[...] + p1_ref[...]

        @pl.when(t == 0)
        def _():
            s_ref[...] = jnp.zeros((1, 2 * F), jnp.float32)

        s1 = jnp.sum(h3, axis=0, keepdims=True)
        s2 = jnp.sum(h3 * h3, axis=0, keepdims=True)
        s_ref[...] += jnp.concatenate([s1, s2], axis=1)

    @pl.when(p == 1)
    def _phase1():
        @pl.when(t == 0)
        def _():
            m_ref[...] = jnp.full((1, G), -jnp.inf, jnp.float32)
            d_ref[...] = jnp.zeros((1, G), jnp.float32)
            P_ref[...] = jnp.zeros((DIM, G), jnp.float32)

        stats = s_ref[...]
        mean = stats[:, 0:F] / N
        var = stats[:, F:2 * F] / N - mean * mean
        rstd = lax.rsqrt(var + 1e-5)

        h3 = p0_ref[...] + p1_ref[...]
        hn = (h3 - mean) * rstd * gam_ref[...] + bet_ref[...]

        z = jnp.maximum(jnp.dot(hn, W1_ref[...], precision=_HIGHEST)
                        + b1_ref[...], 0.0)
        z = jnp.maximum(jnp.dot(z, W2_ref[...], precision=_HIGHEST)
                        + b2_ref[...], 0.0)
        z = jnp.maximum(jnp.dot(z, W3_ref[...], precision=_HIGHEST)
                        + b3_ref[...], 0.0)
        gate = jnp.dot(z, gW_ref[...], precision=_HIGHEST) + gb_ref[...]

        seg = bc_ref[...]
        oh = lax.broadcasted_iota(jnp.int32, (TN, G), 1) == seg
        ohf = oh.astype(jnp.float32)

        neg_inf = jnp.float32(-jnp.inf)
        tile_m = jnp.max(jnp.where(oh, gate, neg_inf), axis=0,
                         keepdims=True)
        m_old = m_ref[...]
        m_new = jnp.maximum(m_old, tile_m)
        m_ref[...] = m_new
        scale = jnp.where(m_new == neg_inf, 0.0, jnp.exp(m_old - m_new))

        m_node = jnp.max(jnp.where(oh, m_new, neg_inf), axis=1,
                         keepdims=True)
        e = jnp.where(m_node == neg_inf, 0.0, jnp.exp(gate - m_node))

        d_ref[...] = d_ref[...] * scale + _dot0(e, ohf)
        P_ref[...] = P_ref[...] * scale + _dot0(e * z, ohf)

        @pl.when(t == T - 1)
        def _():
            pooled_t = P_ref[...] / (d_ref[...] + 1e-16)
            a_row = _dot0(f1W_ref[...], pooled_t) + f1b_ref[...]
            b_row = _dot0(f2W_ref[...], pooled_t) + f2b_ref[...]
            out_a_ref[...] = a_row
            out_b_ref[...] = b_row
            out_o_ref[...] = a_row * base_ref[...] + b_row


def _tc_main(parts, bc, W1p, b1, W2, b2, W3, b3, gW, gb,
             f1W, f1b, f2W, f2b, gam, bet, baser):
    full = lambda shape: pl.BlockSpec(shape, lambda p, t: (0,) * len(shape))
    specs = [
        pl.BlockSpec((TN, F), lambda p, t: (t, 0)),
        pl.BlockSpec((TN, F), lambda p, t: (t + T, 0)),
        pl.BlockSpec((TN, 1), lambda p, t: (t, 0)),
        full((F, DIM)), full((1, DIM)), full((DIM, DIM)), full((1, DIM)),
        full((DIM, DIM)), full((1, DIM)), full((DIM, 1)), full((1, 1)),
        full((DIM, 1)), full((1, 1)), full((DIM, 1)), full((1, 1)),
        full((1, F)), full((1, F)), full((1, G)),
    ]
    out_specs = [full((1, G)), full((1, G)), full((1, G))]
    out_shapes = [jax.ShapeDtypeStruct((1, G), jnp.float32)] * 3
    return pl.pallas_call(
        _tc_body,
        grid=(2, T),
        in_specs=specs,
        out_specs=out_specs,
        out_shape=out_shapes,
        scratch_shapes=[
            pltpu.VMEM((1, 2 * F), jnp.float32),
            pltpu.VMEM((1, G), jnp.float32),
            pltpu.VMEM((1, G), jnp.float32),
            pltpu.VMEM((DIM, G), jnp.float32),
        ],
    )(parts, parts, bc, W1p, b1, W2, b2, W3, b3, gW, gb,
      f1W, f1b, f2W, f2b, gam, bet, baser)


def kernel(x, edge_index, batch, base, bn_gamma, bn_beta, W1, b1, W2, b2,
           W3, b3, gate_W, gate_b, fc1_W, fc1_b, fc2_W, fc2_b):
    E = edge_index.shape[1]
    epw = NW * KB * EROW
    e_pad = ((E + epw - 1) // epw) * epw

    xp = jnp.zeros((N_PAD, F), jnp.float32).at[:N, :3].set(x)
    xhalf = xp * 0.5

    fill = N + (lax.iota(jnp.int32, e_pad - E) % 256)
    srcm = jnp.concatenate([edge_index[0], fill]).reshape(-1, EROW)
    dstm = jnp.concatenate([edge_index[1], fill]).reshape(-1, EROW)

    parts = _sc_agg(xp, xhalf, srcm, dstm)

    bc = jnp.full((N_PAD, 1), G, jnp.int32).at[:N, 0].set(batch)
    W1p = jnp.zeros((F, DIM), jnp.float32).at[:3].set(W1)
    gam = jnp.zeros((1, F), jnp.float32).at[0, :3].set(bn_gamma)
    bet = jnp.zeros((1, F), jnp.float32).at[0, :3].set(bn_beta)

    out_row, a_row, b_row = _tc_main(
        parts, bc, W1p, b1.reshape(1, DIM), W2, b2.reshape(1, DIM),
        W3, b3.reshape(1, DIM), gate_W, gate_b.reshape(1, 1),
        fc1_W, fc1_b.reshape(1, 1), fc2_W, fc2_b.reshape(1, 1),
        gam, bet, base.reshape(1, G))

    return (out_row.reshape(G, 1), a_row.reshape(G, 1),
            b_row.reshape(G, 1))


# speedup vs baseline: 19.2686x; 1.3290x over previous
import functools

import jax
import jax.numpy as jnp
from jax import lax
from jax.experimental import pallas as pl
from jax.experimental.pallas import tpu as pltpu
from jax.experimental.pallas import tpu_sc as plsc

N = 100000
G = 512
DIM = 128
F = 8
TN = 1024
T = 98
N_PAD = T * TN

EROW = 128
KB = 80
BATCH_G = 8
NW = 32
NSUB = 16
ROWS_PER_TILE = N_PAD // NSUB

_HIGHEST = jax.lax.Precision.HIGHEST


def _dot0(a, b):
    return lax.dot_general(a, b, (((0,), (0,)), ((), ())),
                           precision=_HIGHEST)


def _sc_agg(xp, xhalf, srcm, dstm):
    n_rows = srcm.shape[0]
    rows_per_w = n_rows // NW
    nblk = rows_per_w // KB

    mesh = plsc.VectorSubcoreMesh(core_axis_name="c", subcore_axis_name="s")

    @functools.partial(
        pl.kernel,
        out_type=jax.ShapeDtypeStruct((2 * N_PAD, F), jnp.float32),
        mesh=mesh,
        compiler_params=pltpu.CompilerParams(use_tc_tiling_on_sc=False),
        scratch_types=[
            pltpu.VMEM((KB, EROW), jnp.int32),
            pltpu.VMEM((KB, EROW), jnp.int32),
            pltpu.VMEM((BATCH_G, EROW, F), jnp.float32),
            pltpu.VMEM((ROWS_PER_TILE, F), jnp.float32),
            pltpu.VMEM_SHARED((N_PAD, F), jnp.float32),
            pltpu.SemaphoreType.DMA,
        ],
    )
    def k(x_hbm, xh_hbm, src_hbm, dst_hbm, out_hbm,
          sbuf, dbuf, rows, bounce, acc_sh, sem):
        c = lax.axis_index("c")
        s = lax.axis_index("s")
        wid = c * NSUB + s

        r0 = s * ROWS_PER_TILE
        pltpu.sync_copy(xh_hbm.at[pl.ds(r0, ROWS_PER_TILE)], bounce)
        pltpu.sync_copy(bounce, acc_sh.at[pl.ds(r0, ROWS_PER_TILE)])
        plsc.subcore_barrier()

        wbase = wid * rows_per_w

        def blk_body(blk, carry):
            row0 = wbase + blk * KB
            pltpu.sync_copy(src_hbm.at[pl.ds(row0, KB)], sbuf)
            pltpu.sync_copy(dst_hbm.at[pl.ds(row0, KB)], dbuf)

            def edge_body(g, carry2):
                descs = []
                for j in range(BATCH_G):
                    descs.append(pltpu.async_copy(
                        x_hbm.at[sbuf.at[g * BATCH_G + j]],
                        rows.at[j], sem))
                for dsc in descs:
                    dsc.wait()
                for j in range(BATCH_G):
                    pltpu.sync_copy(rows.at[j],
                                    acc_sh.at[dbuf.at[g * BATCH_G + j]],
                                    add=True)
                return carry2

            return lax.fori_loop(0, KB // BATCH_G, edge_body, carry)

        lax.fori_loop(0, nblk, blk_body, 0)
        plsc.subcore_barrier()

        off = c * N_PAD + s * ROWS_PER_TILE
        pltpu.sync_copy(acc_sh.at[pl.ds(r0, ROWS_PER_TILE)], bounce)
        pltpu.sync_copy(bounce, out_hbm.at[pl.ds(off, ROWS_PER_TILE)])

    return k(xp, xhalf, srcm, dstm)


def _tc_body(p0_ref, p1_ref, bc_ref, W1_ref, b1_ref, W2_ref, b2_ref,
             W3_ref, b3_ref, gW_ref, gb_ref, f1W_ref, f1b_ref, f2W_ref,
             f2b_ref, gam_ref, bet_ref, base_ref,
             out_o_ref, out_a_ref, out_b_ref,
             s_ref, m_ref, d_ref, P_ref):
    p = pl.program_id(0)
    t = pl.program_id(1)

    @pl.when(p == 0)
    def _phase0():
        h3 = p0_ref[...] + p1_ref[...]

        @pl.when(t == 0)
        def _():
            s_ref[...] = jnp.zeros((1, 2 * F), jnp.float32)

        s1 = jnp.sum(h3, axis=0, keepdims=True)
        s2 = jnp.sum(h3 * h3, axis=0, keepdims=True)
        s_ref[...] += jnp.concatenate([s1, s2], axis=1)

    @pl.when(p == 1)
    def _phase1():
        @pl.when(t == 0)
        def _():
            m_ref[...] = jnp.full((1, G), -jnp.inf, jnp.float32)
            d_ref[...] = jnp.zeros((1, G), jnp.float32)
            P_ref[...] = jnp.zeros((DIM, G), jnp.float32)

        stats = s_ref[...]
        mean = stats[:, 0:F] / N
        var = stats[:, F:2 * F] / N - mean * mean
        rstd = lax.rsqrt(var + 1e-5)

        h3 = p0_ref[...] + p1_ref[...]
        hn = (h3 - mean) * rstd * gam_ref[...] + bet_ref[...]

        z = jnp.maximum(jnp.dot(hn, W1_ref[...], precision=_HIGHEST)
                        + b1_ref[...], 0.0)
        z = jnp.maximum(jnp.dot(z, W2_ref[...], precision=_HIGHEST)
                        + b2_ref[...], 0.0)
        z = jnp.maximum(jnp.dot(z, W3_ref[...], precision=_HIGHEST)
                        + b3_ref[...], 0.0)
        gate = jnp.dot(z, gW_ref[...], precision=_HIGHEST) + gb_ref[...]

        seg = bc_ref[...]
        oh = lax.broadcasted_iota(jnp.int32, (TN, G), 1) == seg
        ohf = oh.astype(jnp.float32)

        neg_inf = jnp.float32(-jnp.inf)
        tile_m = jnp.max(jnp.where(oh, gate, neg_inf), axis=0,
                         keepdims=True)
        m_old = m_ref[...]
        m_new = jnp.maximum(m_old, tile_m)
        m_ref[...] = m_new
        scale = jnp.where(m_new == neg_inf, 0.0, jnp.exp(m_old - m_new))

        m_node = jnp.max(jnp.where(oh, m_new, neg_inf), axis=1,
                         keepdims=True)
        e = jnp.where(m_node == neg_inf, 0.0, jnp.exp(gate - m_node))

        d_ref[...] = d_ref[...] * scale + _dot0(e, ohf)
        P_ref[...] = P_ref[...] * scale + _dot0(e * z, ohf)

        @pl.when(t == T - 1)
        def _():
            pooled_t = P_ref[...] / (d_ref[...] + 1e-16)
            a_row = _dot0(f1W_ref[...], pooled_t) + f1b_ref[...]
            b_row = _dot0(f2W_ref[...], pooled_t) + f2b_ref[...]
            out_a_ref[...] = a_row
            out_b_ref[...] = b_row
            out_o_ref[...] = a_row * base_ref[...] + b_row


def _tc_main(parts, bc, W1p, b1, W2, b2, W3, b3, gW, gb,
             f1W, f1b, f2W, f2b, gam, bet, baser):
    full = lambda shape: pl.BlockSpec(shape, lambda p, t: (0,) * len(shape))
    specs = [
        pl.BlockSpec((TN, F), lambda p, t: (t, 0)),
        pl.BlockSpec((TN, F), lambda p, t: (t + T, 0)),
        pl.BlockSpec((TN, 1), lambda p, t: (t, 0)),
        full((F, DIM)), full((1, DIM)), full((DIM, DIM)), full((1, DIM)),
        full((DIM, DIM)), full((1, DIM)), full((DIM, 1)), full((1, 1)),
        full((DIM, 1)), full((1, 1)), full((DIM, 1)), full((1, 1)),
        full((1, F)), full((1, F)), full((1, G)),
    ]
    out_specs = [full((1, G)), full((1, G)), full((1, G))]
    out_shapes = [jax.ShapeDtypeStruct((1, G), jnp.float32)] * 3
    return pl.pallas_call(
        _tc_body,
        grid=(2, T),
        in_specs=specs,
        out_specs=out_specs,
        out_shape=out_shapes,
        scratch_shapes=[
            pltpu.VMEM((1, 2 * F), jnp.float32),
            pltpu.VMEM((1, G), jnp.float32),
            pltpu.VMEM((1, G), jnp.float32),
            pltpu.VMEM((DIM, G), jnp.float32),
        ],
    )(parts, parts, bc, W1p, b1, W2, b2, W3, b3, gW, gb,
      f1W, f1b, f2W, f2b, gam, bet, baser)


def kernel(x, edge_index, batch, base, bn_gamma, bn_beta, W1, b1, W2, b2,
           W3, b3, gate_W, gate_b, fc1_W, fc1_b, fc2_W, fc2_b):
    E = edge_index.shape[1]
    epw = NW * KB * EROW
    e_pad = ((E + epw - 1) // epw) * epw

    xp = jnp.zeros((N_PAD, F), jnp.float32).at[:N, :3].set(x)
    xhalf = xp * 0.5

    fill = N + (lax.iota(jnp.int32, e_pad - E) % 256)
    srcm = jnp.concatenate([edge_index[0], fill]).reshape(-1, EROW)
    dstm = jnp.concatenate([edge_index[1], fill]).reshape(-1, EROW)

    parts = _sc_agg(xp, xhalf, srcm, dstm)

    bc = jnp.full((N_PAD, 1), G, jnp.int32).at[:N, 0].set(batch)
    W1p = jnp.zeros((F, DIM), jnp.float32).at[:3].set(W1)
    gam = jnp.zeros((1, F), jnp.float32).at[0, :3].set(bn_gamma)
    bet = jnp.zeros((1, F), jnp.float32).at[0, :3].set(bn_beta)

    out_row, a_row, b_row = _tc_main(
        parts, bc, W1p, b1.reshape(1, DIM), W2, b2.reshape(1, DIM),
        W3, b3.reshape(1, DIM), gate_W, gate_b.reshape(1, 1),
        fc1_W, fc1_b.reshape(1, 1), fc2_W, fc2_b.reshape(1, 1),
        gam, bet, base.reshape(1, G))

    return (out_row.reshape(G, 1), a_row.reshape(G, 1),
            b_row.reshape(G, 1))


# speedup vs baseline: 20.0587x; 1.0410x over previous
import functools

import jax
import jax.numpy as jnp
from jax import lax
from jax.experimental import pallas as pl
from jax.experimental.pallas import tpu as pltpu
from jax.experimental.pallas import tpu_sc as plsc

N = 100000
G = 512
DIM = 128
F = 8
TN = 1024
T = 98
N_PAD = T * TN

EROW = 128
KB = 80
BATCH_G = 8
NW = 32
NSUB = 16
ROWS_PER_TILE = N_PAD // NSUB

_HIGHEST = jax.lax.Precision.HIGHEST


def _dot0(a, b):
    return lax.dot_general(a, b, (((0,), (0,)), ((), ())),
                           precision=_HIGHEST)


def _sc_agg(xp, srcm, dstm):
    n_rows = srcm.shape[0]
    rows_per_w = n_rows // NW
    nblk = rows_per_w // KB

    mesh = plsc.VectorSubcoreMesh(core_axis_name="c", subcore_axis_name="s")

    @functools.partial(
        pl.kernel,
        out_type=jax.ShapeDtypeStruct((2 * N_PAD, F), jnp.float32),
        mesh=mesh,
        compiler_params=pltpu.CompilerParams(use_tc_tiling_on_sc=False),
        scratch_types=[
            pltpu.VMEM((KB, EROW), jnp.int32),
            pltpu.VMEM((KB, EROW), jnp.int32),
            pltpu.VMEM((BATCH_G, EROW, F), jnp.float32),
            pltpu.VMEM((ROWS_PER_TILE, F), jnp.float32),
            pltpu.VMEM_SHARED((N_PAD, F), jnp.float32),
            pltpu.SemaphoreType.DMA,
        ],
    )
    def k(x_hbm, src_hbm, dst_hbm, out_hbm,
          sbuf, dbuf, rows, bounce, acc_sh, sem):
        c = lax.axis_index("c")
        s = lax.axis_index("s")
        wid = c * NSUB + s

        r0 = s * ROWS_PER_TILE
        pltpu.sync_copy(x_hbm.at[pl.ds(r0, ROWS_PER_TILE)], bounce)
        pltpu.sync_copy(bounce, acc_sh.at[pl.ds(r0, ROWS_PER_TILE)])
        plsc.subcore_barrier()

        wbase = wid * rows_per_w

        def blk_body(blk, carry):
            row0 = wbase + blk * KB
            pltpu.sync_copy(src_hbm.at[pl.ds(row0, KB)], sbuf)
            pltpu.sync_copy(dst_hbm.at[pl.ds(row0, KB)], dbuf)

            def edge_body(g, carry2):
                descs = []
                for j in range(BATCH_G):
                    descs.append(pltpu.async_copy(
                        x_hbm.at[sbuf.at[g * BATCH_G + j]],
                        rows.at[j], sem))
                for dsc in descs:
                    dsc.wait()
                for j in range(BATCH_G):
                    pltpu.sync_copy(rows.at[j],
                                    acc_sh.at[dbuf.at[g * BATCH_G + j]],
                                    add=True)
                return carry2

            return lax.fori_loop(0, KB // BATCH_G, edge_body, carry)

        lax.fori_loop(0, nblk, blk_body, 0)
        plsc.subcore_barrier()

        off = c * N_PAD + s * ROWS_PER_TILE
        pltpu.sync_copy(acc_sh.at[pl.ds(r0, ROWS_PER_TILE)], bounce)
        pltpu.sync_copy(bounce, out_hbm.at[pl.ds(off, ROWS_PER_TILE)])

    return k(xp, srcm, dstm)


def _tc_body(p0_ref, p1_ref, xp_ref, lo_ref, hi_ref, W1_ref, b1_ref,
             W2_ref, b2_ref, W3_ref, b3_ref, gW_ref, gb_ref, f1W_ref,
             f1b_ref, f2W_ref, f2b_ref, gam_ref, bet_ref, base_ref,
             out_o_ref, out_a_ref, out_b_ref,
             s_ref, m_ref, d_ref, P_ref):
    p = pl.program_id(0)
    t = pl.program_id(1)

    @pl.when(p == 0)
    def _phase0():
        h3 = p0_ref[...] + p1_ref[...] - xp_ref[...]

        @pl.when(t == 0)
        def _():
            s_ref[...] = jnp.zeros((1, 2 * F), jnp.float32)

        s1 = jnp.sum(h3, axis=0, keepdims=True)
        s2 = jnp.sum(h3 * h3, axis=0, keepdims=True)
        s_ref[...] += jnp.concatenate([s1, s2], axis=1)

    @pl.when(p == 1)
    def _phase1():
        @pl.when(t == 0)
        def _():
            m_ref[...] = jnp.full((1, G), -jnp.inf, jnp.float32)
            d_ref[...] = jnp.zeros((1, G), jnp.float32)
            P_ref[...] = jnp.zeros((DIM, G), jnp.float32)

        stats = s_ref[...]
        mean = stats[:, 0:F] / N
        var = stats[:, F:2 * F] / N - mean * mean
        rstd = lax.rsqrt(var + 1e-5)

        h3 = p0_ref[...] + p1_ref[...] - xp_ref[...]
        hn = (h3 - mean) * rstd * gam_ref[...] + bet_ref[...]

        z = jnp.maximum(jnp.dot(hn, W1_ref[...], precision=_HIGHEST)
                        + b1_ref[...], 0.0)
        z = jnp.maximum(jnp.dot(z, W2_ref[...], precision=_HIGHEST)
                        + b2_ref[...], 0.0)
        z = jnp.maximum(jnp.dot(z, W3_ref[...], precision=_HIGHEST)
                        + b3_ref[...], 0.0)
        gate = jnp.dot(z, gW_ref[...], precision=_HIGHEST) + gb_ref[...]

        gid = t * TN + lax.broadcasted_iota(jnp.int32, (TN, 1), 0)
        oh = jnp.logical_and(gid >= lo_ref[...], gid < hi_ref[...])
        ohf = oh.astype(jnp.float32)

        neg_inf = jnp.float32(-jnp.inf)
        tile_m = jnp.max(jnp.where(oh, gate, neg_inf), axis=0,
                         keepdims=True)
        m_old = m_ref[...]
        m_new = jnp.maximum(m_old, tile_m)
        m_ref[...] = m_new
        scale = jnp.where(m_new == neg_inf, 0.0, jnp.exp(m_old - m_new))

        m_node = jnp.max(jnp.where(oh, m_new, neg_inf), axis=1,
                         keepdims=True)
        e = jnp.where(m_node == neg_inf, 0.0, jnp.exp(gate - m_node))

        d_ref[...] = d_ref[...] * scale + _dot0(e, ohf)
        P_ref[...] = P_ref[...] * scale + _dot0(e * z, ohf)

        @pl.when(t == T - 1)
        def _():
            pooled_t = P_ref[...] / (d_ref[...] + 1e-16)
            a_row = _dot0(f1W_ref[...], pooled_t) + f1b_ref[...]
            b_row = _dot0(f2W_ref[...], pooled_t) + f2b_ref[...]
            out_a_ref[...] = a_row
            out_b_ref[...] = b_row
            out_o_ref[...] = a_row * base_ref[...] + b_row


def _tc_main(parts, xp, lo, hi, W1p, b1, W2, b2, W3, b3, gW, gb,
             f1W, f1b, f2W, f2b, gam, bet, baser):
    full = lambda shape: pl.BlockSpec(shape, lambda p, t: (0,) * len(shape))
    specs = [
        pl.BlockSpec((TN, F), lambda p, t: (t, 0)),
        pl.BlockSpec((TN, F), lambda p, t: (t + T, 0)),
        pl.BlockSpec((TN, F), lambda p, t: (t, 0)),
        full((1, G)), full((1, G)),
        full((F, DIM)), full((1, DIM)), full((DIM, DIM)), full((1, DIM)),
        full((DIM, DIM)), full((1, DIM)), full((DIM, 1)), full((1, 1)),
        full((DIM, 1)), full((1, 1)), full((DIM, 1)), full((1, 1)),
        full((1, F)), full((1, F)), full((1, G)),
    ]
    out_specs = [full((1, G)), full((1, G)), full((1, G))]
    out_shapes = [jax.ShapeDtypeStruct((1, G), jnp.float32)] * 3
    return pl.pallas_call(
        _tc_body,
        grid=(2, T),
        in_specs=specs,
        out_specs=out_specs,
        out_shape=out_shapes,
        scratch_shapes=[
            pltpu.VMEM((1, 2 * F), jnp.float32),
            pltpu.VMEM((1, G), jnp.float32),
            pltpu.VMEM((1, G), jnp.float32),
            pltpu.VMEM((DIM, G), jnp.float32),
        ],
    )(parts, parts, xp, lo, hi, W1p, b1, W2, b2, W3, b3, gW, gb,
      f1W, f1b, f2W, f2b, gam, bet, baser)


def kernel(x, edge_index, batch, base, bn_gamma, bn_beta, W1, b1, W2, b2,
           W3, b3, gate_W, gate_b, fc1_W, fc1_b, fc2_W, fc2_b):
    E = edge_index.shape[1]
    epw = NW * KB * EROW
    e_pad = ((E + epw - 1) // epw) * epw

    xp = jnp.zeros((N_PAD, F), jnp.float32).at[:N, :3].set(x)

    fill = N + (lax.iota(jnp.int32, e_pad - E) % 256)
    srcm = jnp.concatenate([edge_index[0], fill]).reshape(-1, EROW)
    dstm = jnp.concatenate([edge_index[1], fill]).reshape(-1, EROW)

    parts = _sc_agg(xp, srcm, dstm)

    starts = jnp.searchsorted(batch, jnp.arange(G + 1, dtype=jnp.int32)
                              ).astype(jnp.int32)
    lo = starts[:G].reshape(1, G)
    hi = starts[1:].reshape(1, G)
    W1p = jnp.zeros((F, DIM), jnp.float32).at[:3].set(W1)
    gam = jnp.zeros((1, F), jnp.float32).at[0, :3].set(bn_gamma)
    bet = jnp.zeros((1, F), jnp.float32).at[0, :3].set(bn_beta)

    out_row, a_row, b_row = _tc_main(
        parts, xp, lo, hi, W1p, b1.reshape(1, DIM), W2, b2.reshape(1, DIM),
        W3, b3.reshape(1, DIM), gate_W, gate_b.reshape(1, 1),
        fc1_W, fc1_b.reshape(1, 1), fc2_W, fc2_b.reshape(1, 1),
        gam, bet, base.reshape(1, G))

    return (out_row.reshape(G, 1), a_row.reshape(G, 1),
            b_row.reshape(G, 1))
